# baseline (speedup 1.0000x reference)
import jax
import jax.numpy as jnp
from jax import lax
from jax.experimental import pallas as pl
from jax.experimental.pallas import tpu as pltpu
from jax.experimental.pallas import tpu_sc as plsc

N = 10000
E = 320000
D = 128

NC = 2
NS = 16
NW = NC * NS
K = 128
EPT = E // NW
STEPS = 80
NP = 10240
RPT = NP // NS
TRASH = 10016
SHIFT = 14

_MESH = plsc.VectorSubcoreMesh(core_axis_name="c", subcore_axis_name="s")


def _unpack(packed_v, j, dst_ref, p, src_ref=None):
    for k in range(8):
        w = packed_v[j, pl.ds(16 * k, 16)]
        d = lax.shift_right_logical(w, SHIFT)
        dst_ref[p, pl.ds(16 * k, 16)] = d
        if src_ref is not None:
            src_ref[p, pl.ds(16 * k, 16)] = w - (d << SHIFT)


def _cnt_body(pk3_hbm, ones_hbm, z16_hbm, out_hbm, cacc, pk_v, ones_v, dstb,
              ssem):
    c = lax.axis_index("c")
    s = lax.axis_index("s")
    wid = c * NS + s
    pltpu.sync_copy(z16_hbm, cacc.at[pl.ds(s * RPT, RPT)])
    pltpu.sync_copy(ones_hbm, ones_v)
    pltpu.sync_copy(pk3_hbm.at[wid], pk_v)
    plsc.subcore_barrier()

    def step(i, carry):
        p = lax.rem(i, 2)
        for b in range(2):
            j = 2 * i + b
            _unpack(pk_v, j, dstb[b], p)
            pltpu.async_copy(ones_v, cacc.at[dstb[b].at[p]], ssem[b],
                             add=True)

            @pl.when(j + 2 < STEPS)
            def _():
                pltpu.make_async_copy(
                    ones_v, cacc.at[dstb[b].at[p]], ssem[b]).wait()

            del _
        return carry

    lax.fori_loop(0, STEPS // 2, step, 0)
    for b in range(2):
        pltpu.make_async_copy(ones_v, cacc.at[dstb[b].at[0]], ssem[b]).wait()
    plsc.subcore_barrier()
    pltpu.sync_copy(cacc.at[pl.ds(s * RPT, RPT)],
                    out_hbm.at[c, pl.ds(s * RPT, RPT)])


_cnt_kernel = pl.kernel(
    _cnt_body,
    out_type=jax.ShapeDtypeStruct((NC, NP, 16), jnp.float32),
    mesh=_MESH,
    scratch_types=[
        pltpu.VMEM_SHARED((NP, 16), jnp.float32),
        pltpu.VMEM((STEPS, K), jnp.int32),
        pltpu.VMEM((K, 16), jnp.float32),
        [pltpu.VMEM((2, K), jnp.int32)] * 2,
        [pltpu.SemaphoreType.DMA] * 2,
    ],
    compiler_params=pltpu.CompilerParams(use_tc_tiling_on_sc=False),
)


def _edge_body(y_hbm, pk3_hbm, zrows_hbm, out_hbm, zacc, pk_v, srcb, dstb,
               rows, gsem, ssem):
    c = lax.axis_index("c")
    s = lax.axis_index("s")
    wid = c * NS + s
    pltpu.sync_copy(zrows_hbm, zacc.at[pl.ds(s * RPT, RPT)])
    pltpu.sync_copy(pk3_hbm.at[wid], pk_v)
    plsc.subcore_barrier()

    def gather_start(j, b, p):
        _unpack(pk_v, j, dstb[b], p, srcb[b])
        pltpu.async_copy(y_hbm.at[srcb[b].at[p]], rows[b], gsem[b])

    def gather_wait(b):
        pltpu.make_async_copy(y_hbm.at[srcb[b].at[0]], rows[b],
                              gsem[b]).wait()

    def scatter_start(b, p):
        pltpu.async_copy(rows[b], zacc.at[dstb[b].at[p]], ssem[b], add=True)

    def scatter_wait(b, p):
        pltpu.make_async_copy(rows[b], zacc.at[dstb[b].at[p]], ssem[b]).wait()

    for b in range(2):
        gather_start(b, b, 0)

    def step(i, carry):
        p = lax.rem(i, 2)
        for b in range(2):
            j = 2 * i + b
            gather_wait(b)
            scatter_start(b, p)

            @pl.when(j + 2 < STEPS)
            def _():
                scatter_wait(b, p)
                gather_start(j + 2, b, 1 - p)

            del _
        return carry

    lax.fori_loop(0, STEPS // 2, step, 0)
    for b in range(2):
        scatter_wait(b, 0)
    plsc.subcore_barrier()
    pltpu.sync_copy(zacc.at[pl.ds(s * RPT, RPT)],
                    out_hbm.at[c, pl.ds(s * RPT, RPT)])


_edge_kernel = pl.kernel(
    _edge_body,
    out_type=jax.ShapeDtypeStruct((NC, NP, D), jnp.float32),
    mesh=_MESH,
    scratch_types=[
        pltpu.VMEM_SHARED((NP, D), jnp.float32),
        pltpu.VMEM((STEPS, K), jnp.int32),
        [pltpu.VMEM((2, K), jnp.int32)] * 2,
        [pltpu.VMEM((2, K), jnp.int32)] * 2,
        [pltpu.VMEM((K, D), jnp.float32)] * 2,
        [pltpu.SemaphoreType.DMA] * 2,
        [pltpu.SemaphoreType.DMA] * 2,
    ],
    compiler_params=pltpu.CompilerParams(use_tc_tiling_on_sc=False),
)


BN = 1000


def _dinv(cnt_ref):
    deg = cnt_ref[0, :, 0:1] + cnt_ref[1, :, 0:1] + 1.0
    return lax.rsqrt(deg)


def _k1_body(cnt_ref, x_ref, w_ref, y_ref):
    y_ref[...] = _dinv(cnt_ref) * jnp.dot(
        x_ref[...], w_ref[...], preferred_element_type=jnp.float32)


def _k2_body(cnt_ref, z_ref, y_ref, b_ref, w_ref, o_ref):
    dinv = _dinv(cnt_ref)
    h = jnp.maximum(
        dinv * (z_ref[0] + z_ref[1] + y_ref[...]) + b_ref[...], 0.0)
    o_ref[...] = dinv * jnp.dot(
        h, w_ref[...], preferred_element_type=jnp.float32)


def _k3_body(cnt_ref, z_ref, y_ref, b_ref, o_ref):
    o_ref[...] = (_dinv(cnt_ref) * (z_ref[0] + z_ref[1] + y_ref[...])
                  + b_ref[...])


_cnt_spec = pl.BlockSpec((NC, BN, 16), lambda i: (0, i, 0))
_row_spec = pl.BlockSpec((BN, D), lambda i: (i, 0))
_z_spec = pl.BlockSpec((NC, BN, D), lambda i: (0, i, 0))
_w_spec = pl.BlockSpec((D, D), lambda i: (0, 0))
_b_spec = pl.BlockSpec((1, D), lambda i: (0, 0))
_out_shape = jax.ShapeDtypeStruct((N, D), jnp.float32)

_k1 = pl.pallas_call(
    _k1_body, grid=(N // BN,),
    in_specs=[_cnt_spec, _row_spec, _w_spec],
    out_specs=_row_spec, out_shape=_out_shape)

_k2 = pl.pallas_call(
    _k2_body, grid=(N // BN,),
    in_specs=[_cnt_spec, _z_spec, _row_spec, _b_spec, _w_spec],
    out_specs=_row_spec, out_shape=_out_shape)

_k3 = pl.pallas_call(
    _k3_body, grid=(N // BN,),
    in_specs=[_cnt_spec, _z_spec, _row_spec, _b_spec],
    out_specs=_row_spec, out_shape=_out_shape)


def kernel(x, edge_index, W1, b1, W2, b2):
    packed = edge_index[0] + (edge_index[1] << SHIFT)
    pad = jnp.full((NW, STEPS * K - EPT), TRASH << SHIFT, jnp.int32)
    pk3 = jnp.concatenate(
        [packed.reshape(NW, EPT), pad], axis=1).reshape(NW, STEPS, K)
    ones16 = jnp.ones((K, 16), jnp.float32)
    z16 = jnp.zeros((RPT, 16), jnp.float32)
    zrows = jnp.zeros((RPT, D), jnp.float32)

    cnt = _cnt_kernel(pk3, ones16, z16)
    y1 = _k1(cnt, x, W1)
    z1 = _edge_kernel(y1, pk3, zrows)
    y2 = _k2(cnt, z1, y1, b1.reshape(1, D), W2)
    z2 = _edge_kernel(y2, pk3, zrows)
    return _k3(cnt, z2, y2, b2.reshape(1, D))


# speedup vs baseline: 2.9915x; 2.9915x over previous
import jax
import jax.numpy as jnp
from jax import lax
from jax.experimental import pallas as pl
from jax.experimental.pallas import tpu as pltpu
from jax.experimental.pallas import tpu_sc as plsc

N = 10000
E = 320000
D = 128

NC = 2
NS = 16
NW = NC * NS
K = 40
EPT = E // NW
STEPS = EPT // K
NBUF = 5
NP = 10240
RPT = NP // NS

_MESH = plsc.VectorSubcoreMesh(core_axis_name="c", subcore_axis_name="s")


def _cnt_body(dst2_hbm, ones_hbm, z16_hbm, out_hbm, cacc, dst_f, ones_v,
              ssem):
    c = lax.axis_index("c")
    s = lax.axis_index("s")
    wid = c * NS + s
    pltpu.sync_copy(z16_hbm, cacc.at[pl.ds(s * RPT, RPT)])
    pltpu.sync_copy(ones_hbm, ones_v)
    pltpu.sync_copy(dst2_hbm.at[wid], dst_f)
    plsc.subcore_barrier()

    def scatter_start(j, b):
        pltpu.async_copy(ones_v, cacc.at[dst_f.at[pl.ds(j * K, K)]], ssem[b],
                         add=True)

    def scatter_wait(b):
        pltpu.make_async_copy(ones_v, cacc.at[dst_f.at[pl.ds(0, K)]],
                              ssem[b]).wait()

    def step(i, carry):
        for b in range(NBUF):
            j = i * NBUF + b
            scatter_start(j, b)

            @pl.when(j + NBUF < STEPS)
            def _():
                scatter_wait(b)

            del _
        return carry

    lax.fori_loop(0, STEPS // NBUF, step, 0)
    for b in range(NBUF):
        scatter_wait(b)
    plsc.subcore_barrier()
    pltpu.sync_copy(cacc.at[pl.ds(s * RPT, RPT)],
                    out_hbm.at[c, pl.ds(s * RPT, RPT)])


_cnt_kernel = pl.kernel(
    _cnt_body,
    out_type=jax.ShapeDtypeStruct((NC, NP, 16), jnp.float32),
    mesh=_MESH,
    scratch_types=[
        pltpu.VMEM_SHARED((NP, 16), jnp.float32),
        pltpu.VMEM((EPT,), jnp.int32),
        pltpu.VMEM((K, 16), jnp.float32),
        [pltpu.SemaphoreType.DMA] * NBUF,
    ],
    compiler_params=pltpu.CompilerParams(use_tc_tiling_on_sc=False),
)


def _edge_body(y_hbm, src2_hbm, dst2_hbm, zrows_hbm, out_hbm, zacc, src_f,
               dst_f, rows, gsem, ssem):
    c = lax.axis_index("c")
    s = lax.axis_index("s")
    wid = c * NS + s
    pltpu.sync_copy(zrows_hbm, zacc.at[pl.ds(s * RPT, RPT)])
    pltpu.sync_copy(src2_hbm.at[wid], src_f)
    pltpu.sync_copy(dst2_hbm.at[wid], dst_f)
    plsc.subcore_barrier()

    def gather_start(j, b):
        pltpu.async_copy(y_hbm.at[src_f.at[pl.ds(j * K, K)]], rows[b],
                         gsem[b])

    def gather_wait(j, b):
        pltpu.make_async_copy(y_hbm.at[src_f.at[pl.ds(j * K, K)]], rows[b],
                              gsem[b]).wait()

    def scatter_start(j, b):
        pltpu.async_copy(rows[b], zacc.at[dst_f.at[pl.ds(j * K, K)]],
                         ssem[b], add=True)

    def scatter_wait(j, b):
        pltpu.make_async_copy(rows[b], zacc.at[dst_f.at[pl.ds(j * K, K)]],
                              ssem[b]).wait()

    for b in range(NBUF):
        gather_start(b, b)

    def step(i, carry):
        j0 = i * NBUF
        for b in range(NBUF):
            j = j0 + b
            gather_wait(j, b)
            scatter_start(j, b)

            @pl.when(j + NBUF < STEPS)
            def _():
                scatter_wait(j, b)
                gather_start(j + NBUF, b)

            del _
        return carry

    lax.fori_loop(0, STEPS // NBUF, step, 0)
    for b in range(NBUF):
        scatter_wait(STEPS - NBUF + b, b)
    plsc.subcore_barrier()
    pltpu.sync_copy(zacc.at[pl.ds(s * RPT, RPT)],
                    out_hbm.at[c, pl.ds(s * RPT, RPT)])


_edge_kernel = pl.kernel(
    _edge_body,
    out_type=jax.ShapeDtypeStruct((NC, NP, D), jnp.float32),
    mesh=_MESH,
    scratch_types=[
        pltpu.VMEM_SHARED((NP, D), jnp.float32),
        pltpu.VMEM((EPT,), jnp.int32),
        pltpu.VMEM((EPT,), jnp.int32),
        [pltpu.VMEM((K, D), jnp.float32)] * NBUF,
        [pltpu.SemaphoreType.DMA] * NBUF,
        [pltpu.SemaphoreType.DMA] * NBUF,
    ],
    compiler_params=pltpu.CompilerParams(use_tc_tiling_on_sc=False),
)


BN = 1000


def _dinv(cnt_ref):
    deg = cnt_ref[0, :, 0:1] + cnt_ref[1, :, 0:1] + 1.0
    return lax.rsqrt(deg)


def _k1_body(cnt_ref, x_ref, w_ref, y_ref):
    y_ref[...] = _dinv(cnt_ref) * jnp.dot(
        x_ref[...], w_ref[...], preferred_element_type=jnp.float32)


def _k2_body(cnt_ref, z_ref, y_ref, b_ref, w_ref, o_ref):
    dinv = _dinv(cnt_ref)
    h = jnp.maximum(
        dinv * (z_ref[0] + z_ref[1] + y_ref[...]) + b_ref[...], 0.0)
    o_ref[...] = dinv * jnp.dot(
        h, w_ref[...], preferred_element_type=jnp.float32)


def _k3_body(cnt_ref, z_ref, y_ref, b_ref, o_ref):
    o_ref[...] = (_dinv(cnt_ref) * (z_ref[0] + z_ref[1] + y_ref[...])
                  + b_ref[...])


_cnt_spec = pl.BlockSpec((NC, BN, 16), lambda i: (0, i, 0))
_row_spec = pl.BlockSpec((BN, D), lambda i: (i, 0))
_z_spec = pl.BlockSpec((NC, BN, D), lambda i: (0, i, 0))
_w_spec = pl.BlockSpec((D, D), lambda i: (0, 0))
_b_spec = pl.BlockSpec((1, D), lambda i: (0, 0))
_out_shape = jax.ShapeDtypeStruct((N, D), jnp.float32)

_k1 = pl.pallas_call(
    _k1_body, grid=(N // BN,),
    in_specs=[_cnt_spec, _row_spec, _w_spec],
    out_specs=_row_spec, out_shape=_out_shape)

_k2 = pl.pallas_call(
    _k2_body, grid=(N // BN,),
    in_specs=[_cnt_spec, _z_spec, _row_spec, _b_spec, _w_spec],
    out_specs=_row_spec, out_shape=_out_shape)

_k3 = pl.pallas_call(
    _k3_body, grid=(N // BN,),
    in_specs=[_cnt_spec, _z_spec, _row_spec, _b_spec],
    out_specs=_row_spec, out_shape=_out_shape)


def kernel(x, edge_index, W1, b1, W2, b2):
    src2 = edge_index[0].reshape(NW, EPT)
    dst2 = edge_index[1].reshape(NW, EPT)
    ones16 = jnp.ones((K, 16), jnp.float32)
    z16 = jnp.zeros((RPT, 16), jnp.float32)
    zrows = jnp.zeros((RPT, D), jnp.float32)

    cnt = _cnt_kernel(dst2, ones16, z16)
    y1 = _k1(cnt, x, W1)
    z1 = _edge_kernel(y1, src2, dst2, zrows)
    y2 = _k2(cnt, z1, y1, b1.reshape(1, D), W2)
    z2 = _edge_kernel(y2, src2, dst2, zrows)
    return _k3(cnt, z2, y2, b2.reshape(1, D))


# speedup vs baseline: 3.0566x; 1.0218x over previous
import jax
import jax.numpy as jnp
from jax import lax
from jax.experimental import pallas as pl
from jax.experimental.pallas import tpu as pltpu
from jax.experimental.pallas import tpu_sc as plsc

N = 10000
E = 320000
D = 128

NC = 2
NS = 16
NW = NC * NS
K = 40
EPT = E // NW
STEPS = EPT // K
NBUF = 5
NP = 10240
RPT = NP // NS

_MESH = plsc.VectorSubcoreMesh(core_axis_name="c", subcore_axis_name="s")


def _cnt_body(dst2_hbm, ones_hbm, z16_hbm, out_hbm, cacc, dst_f, ones_v,
              ssem):
    c = lax.axis_index("c")
    s = lax.axis_index("s")
    wid = c * NS + s
    pltpu.sync_copy(z16_hbm, cacc.at[pl.ds(s * RPT, RPT)])
    pltpu.sync_copy(ones_hbm, ones_v)
    pltpu.sync_copy(dst2_hbm.at[wid], dst_f)
    plsc.subcore_barrier()

    def scatter_start(j, b):
        pltpu.async_copy(ones_v, cacc.at[dst_f.at[pl.ds(j * K, K)]], ssem[b],
                         add=True)

    def scatter_wait(b):
        pltpu.make_async_copy(ones_v, cacc.at[dst_f.at[pl.ds(0, K)]],
                              ssem[b]).wait()

    def step(i, carry):
        for b in range(NBUF):
            j = i * NBUF + b
            scatter_start(j, b)

            @pl.when(j + NBUF < STEPS)
            def _():
                scatter_wait(b)

            del _
        return carry

    lax.fori_loop(0, STEPS // NBUF, step, 0)
    for b in range(NBUF):
        scatter_wait(b)
    plsc.subcore_barrier()
    pltpu.sync_copy(cacc.at[pl.ds(s * RPT, RPT)],
                    out_hbm.at[c, pl.ds(s * RPT, RPT)])


_cnt_kernel = pl.kernel(
    _cnt_body,
    out_type=jax.ShapeDtypeStruct((NC, NP, 16), jnp.float32),
    mesh=_MESH,
    scratch_types=[
        pltpu.VMEM_SHARED((NP, 16), jnp.float32),
        pltpu.VMEM((EPT,), jnp.int32),
        pltpu.VMEM((K, 16), jnp.float32),
        [pltpu.SemaphoreType.DMA] * NBUF,
    ],
    compiler_params=pltpu.CompilerParams(use_tc_tiling_on_sc=False),
)


def _edge_body(y_hbm, src2_hbm, dst2_hbm, zrows_hbm, out_hbm, zacc, src_f,
               dst_f, rows, gsem, ssem):
    c = lax.axis_index("c")
    s = lax.axis_index("s")
    wid = c * NS + s
    pltpu.sync_copy(zrows_hbm, zacc.at[pl.ds(s * RPT, RPT)])
    pltpu.sync_copy(src2_hbm.at[wid], src_f)
    pltpu.sync_copy(dst2_hbm.at[wid], dst_f)
    plsc.subcore_barrier()

    def gather_start(j, b):
        pltpu.async_copy(y_hbm.at[src_f.at[pl.ds(j * K, K)]], rows[b],
                         gsem[b])

    def gather_wait(j, b):
        pltpu.make_async_copy(y_hbm.at[src_f.at[pl.ds(j * K, K)]], rows[b],
                              gsem[b]).wait()

    def scatter_start(j, b):
        pltpu.async_copy(rows[b], zacc.at[dst_f.at[pl.ds(j * K, K)]],
                         ssem[b], add=True)

    def scatter_wait(j, b):
        pltpu.make_async_copy(rows[b], zacc.at[dst_f.at[pl.ds(j * K, K)]],
                              ssem[b]).wait()

    for b in range(NBUF):
        gather_start(b, b)

    def step(i, carry):
        j0 = i * NBUF
        for b in range(NBUF):
            j = j0 + b
            gather_wait(j, b)
            scatter_start(j, b)

            @pl.when(j + NBUF < STEPS)
            def _():
                scatter_wait(j, b)
                gather_start(j + NBUF, b)

            del _
        return carry

    lax.fori_loop(0, STEPS // NBUF, step, 0)
    for b in range(NBUF):
        scatter_wait(STEPS - NBUF + b, b)
    plsc.subcore_barrier()
    pltpu.sync_copy(zacc.at[pl.ds(s * RPT, RPT)],
                    out_hbm.at[c, pl.ds(s * RPT, RPT)])


_edge_kernel = pl.kernel(
    _edge_body,
    out_type=jax.ShapeDtypeStruct((NC, NP, D), jnp.float32),
    mesh=_MESH,
    scratch_types=[
        pltpu.VMEM_SHARED((NP, D), jnp.float32),
        pltpu.VMEM((EPT,), jnp.int32),
        pltpu.VMEM((EPT,), jnp.int32),
        [pltpu.VMEM((K, D), jnp.float32)] * NBUF,
        [pltpu.SemaphoreType.DMA] * NBUF,
        [pltpu.SemaphoreType.DMA] * NBUF,
    ],
    compiler_params=pltpu.CompilerParams(use_tc_tiling_on_sc=False),
)


BN = 2000


def _dinv(cnt_ref):
    deg = cnt_ref[0, :, 0:1] + cnt_ref[1, :, 0:1] + 1.0
    return lax.rsqrt(deg)


def _k1_body(cnt_ref, x_ref, w_ref, y_ref):
    y_ref[...] = _dinv(cnt_ref) * jnp.dot(
        x_ref[...], w_ref[...], preferred_element_type=jnp.float32)


def _k2_body(cnt_ref, z_ref, y_ref, b_ref, w_ref, o_ref):
    dinv = _dinv(cnt_ref)
    h = jnp.maximum(
        dinv * (z_ref[0] + z_ref[1] + y_ref[...]) + b_ref[...], 0.0)
    o_ref[...] = dinv * jnp.dot(
        h, w_ref[...], preferred_element_type=jnp.float32)


def _k3_body(cnt_ref, z_ref, y_ref, b_ref, o_ref):
    o_ref[...] = (_dinv(cnt_ref) * (z_ref[0] + z_ref[1] + y_ref[...])
                  + b_ref[...])


_cnt_spec = pl.BlockSpec((NC, BN, 16), lambda i: (0, i, 0))
_row_spec = pl.BlockSpec((BN, D), lambda i: (i, 0))
_z_spec = pl.BlockSpec((NC, BN, D), lambda i: (0, i, 0))
_w_spec = pl.BlockSpec((D, D), lambda i: (0, 0))
_b_spec = pl.BlockSpec((1, D), lambda i: (0, 0))
_out_shape = jax.ShapeDtypeStruct((N, D), jnp.float32)

_k1 = pl.pallas_call(
    _k1_body, grid=(N // BN,),
    in_specs=[_cnt_spec, _row_spec, _w_spec],
    out_specs=_row_spec, out_shape=_out_shape)

_k2 = pl.pallas_call(
    _k2_body, grid=(N // BN,),
    in_specs=[_cnt_spec, _z_spec, _row_spec, _b_spec, _w_spec],
    out_specs=_row_spec, out_shape=_out_shape)

_k3 = pl.pallas_call(
    _k3_body, grid=(N // BN,),
    in_specs=[_cnt_spec, _z_spec, _row_spec, _b_spec],
    out_specs=_row_spec, out_shape=_out_shape)


def kernel(x, edge_index, W1, b1, W2, b2):
    src2 = edge_index[0].reshape(NW, EPT)
    dst2 = edge_index[1].reshape(NW, EPT)
    ones16 = jnp.ones((K, 16), jnp.float32)
    z16 = jnp.zeros((RPT, 16), jnp.float32)
    zrows = jnp.zeros((RPT, D), jnp.float32)

    cnt = _cnt_kernel(dst2, ones16, z16)
    y1 = _k1(cnt, x, W1)
    z1 = _edge_kernel(y1, src2, dst2, zrows)
    y2 = _k2(cnt, z1, y1, b1.reshape(1, D), W2)
    z2 = _edge_kernel(y2, src2, dst2, zrows)
    return _k3(cnt, z2, y2, b2.reshape(1, D))


# speedup vs baseline: 3.1881x; 1.0430x over previous
import jax
import jax.numpy as jnp
from jax import lax
from jax.experimental import pallas as pl
from jax.experimental.pallas import tpu as pltpu
from jax.experimental.pallas import tpu_sc as plsc

N = 10000
E = 320000
D = 128

NC = 2
NS = 16
NW = NC * NS
K = 40
EPT = E // NW
STEPS = EPT // K
NBUF = 5
NBUFC = 10
NP = 10240
RPT = NP // NS

_MESH = plsc.VectorSubcoreMesh(core_axis_name="c", subcore_axis_name="s")


def _cnt_body(ei_hbm, ones_hbm, z16_hbm, out_hbm, cacc, dst_f, ones_v,
              ssem):
    c = lax.axis_index("c")
    s = lax.axis_index("s")
    wid = c * NS + s
    pltpu.sync_copy(z16_hbm, cacc.at[pl.ds(s * RPT, RPT)])
    pltpu.sync_copy(ones_hbm, ones_v)
    pltpu.sync_copy(ei_hbm.at[1, pl.ds(wid * EPT, EPT)], dst_f)
    plsc.subcore_barrier()

    def scatter_start(j, b):
        pltpu.async_copy(ones_v, cacc.at[dst_f.at[pl.ds(j * K, K)]], ssem[b],
                         add=True)

    def scatter_wait(b):
        pltpu.make_async_copy(ones_v, cacc.at[dst_f.at[pl.ds(0, K)]],
                              ssem[b]).wait()

    def step(i, carry):
        for b in range(NBUFC):
            j = i * NBUFC + b
            scatter_start(j, b)

            @pl.when(j + NBUFC < STEPS)
            def _():
                scatter_wait(b)

            del _
        return carry

    lax.fori_loop(0, STEPS // NBUFC, step, 0)
    for b in range(NBUFC):
        scatter_wait(b)
    plsc.subcore_barrier()
    pltpu.sync_copy(cacc.at[pl.ds(s * RPT, RPT)],
                    out_hbm.at[c, pl.ds(s * RPT, RPT)])


_cnt_kernel = pl.kernel(
    _cnt_body,
    out_type=jax.ShapeDtypeStruct((NC, NP, 16), jnp.float32),
    mesh=_MESH,
    scratch_types=[
        pltpu.VMEM_SHARED((NP, 16), jnp.float32),
        pltpu.VMEM((EPT,), jnp.int32),
        pltpu.VMEM((K, 16), jnp.float32),
        [pltpu.SemaphoreType.DMA] * NBUFC,
    ],
    compiler_params=pltpu.CompilerParams(use_tc_tiling_on_sc=False),
)


def _edge_body(y_hbm, ei_hbm, zrows_hbm, out_hbm, zacc, src_f, dst_f, rows,
               gsem, ssem):
    c = lax.axis_index("c")
    s = lax.axis_index("s")
    wid = c * NS + s
    pltpu.sync_copy(zrows_hbm, zacc.at[pl.ds(s * RPT, RPT)])
    pltpu.sync_copy(ei_hbm.at[0, pl.ds(wid * EPT, EPT)], src_f)
    pltpu.sync_copy(ei_hbm.at[1, pl.ds(wid * EPT, EPT)], dst_f)
    plsc.subcore_barrier()

    def gather_start(j, b):
        pltpu.async_copy(y_hbm.at[src_f.at[pl.ds(j * K, K)]], rows[b],
                         gsem[b])

    def gather_wait(j, b):
        pltpu.make_async_copy(y_hbm.at[src_f.at[pl.ds(j * K, K)]], rows[b],
                              gsem[b]).wait()

    def scatter_start(j, b):
        pltpu.async_copy(rows[b], zacc.at[dst_f.at[pl.ds(j * K, K)]],
                         ssem[b], add=True)

    def scatter_wait(j, b):
        pltpu.make_async_copy(rows[b], zacc.at[dst_f.at[pl.ds(j * K, K)]],
                              ssem[b]).wait()

    for b in range(NBUF):
        gather_start(b, b)

    def step(i, carry):
        j0 = i * NBUF
        for b in range(NBUF):
            j = j0 + b
            gather_wait(j, b)
            scatter_start(j, b)

            @pl.when(j + NBUF < STEPS)
            def _():
                scatter_wait(j, b)
                gather_start(j + NBUF, b)

            del _
        return carry

    lax.fori_loop(0, STEPS // NBUF, step, 0)
    for b in range(NBUF):
        scatter_wait(STEPS - NBUF + b, b)
    plsc.subcore_barrier()
    pltpu.sync_copy(zacc.at[pl.ds(s * RPT, RPT)],
                    out_hbm.at[c, pl.ds(s * RPT, RPT)])


_edge_kernel = pl.kernel(
    _edge_body,
    out_type=jax.ShapeDtypeStruct((NC, NP, D), jnp.float32),
    mesh=_MESH,
    scratch_types=[
        pltpu.VMEM_SHARED((NP, D), jnp.float32),
        pltpu.VMEM((EPT,), jnp.int32),
        pltpu.VMEM((EPT,), jnp.int32),
        [pltpu.VMEM((K, D), jnp.float32)] * NBUF,
        [pltpu.SemaphoreType.DMA] * NBUF,
        [pltpu.SemaphoreType.DMA] * NBUF,
    ],
    compiler_params=pltpu.CompilerParams(use_tc_tiling_on_sc=False),
)


BN = 2000


def _dinv(cnt_ref):
    deg = cnt_ref[0, :, 0:1] + cnt_ref[1, :, 0:1] + 1.0
    return lax.rsqrt(deg)


def _k1_body(cnt_ref, x_ref, w_ref, y_ref):
    y_ref[...] = _dinv(cnt_ref) * jnp.dot(
        x_ref[...], w_ref[...], preferred_element_type=jnp.float32)


def _k2_body(cnt_ref, z_ref, y_ref, b_ref, w_ref, o_ref):
    dinv = _dinv(cnt_ref)
    h = jnp.maximum(
        dinv * (z_ref[0] + z_ref[1] + y_ref[...]) + b_ref[...], 0.0)
    o_ref[...] = dinv * jnp.dot(
        h, w_ref[...], preferred_element_type=jnp.float32)


def _k3_body(cnt_ref, z_ref, y_ref, b_ref, o_ref):
    o_ref[...] = (_dinv(cnt_ref) * (z_ref[0] + z_ref[1] + y_ref[...])
                  + b_ref[...])


_cnt_spec = pl.BlockSpec((NC, BN, 16), lambda i: (0, i, 0))
_row_spec = pl.BlockSpec((BN, D), lambda i: (i, 0))
_z_spec = pl.BlockSpec((NC, BN, D), lambda i: (0, i, 0))
_w_spec = pl.BlockSpec((D, D), lambda i: (0, 0))
_b_spec = pl.BlockSpec((1, D), lambda i: (0, 0))
_out_shape = jax.ShapeDtypeStruct((N, D), jnp.float32)

_k1 = pl.pallas_call(
    _k1_body, grid=(N // BN,),
    in_specs=[_cnt_spec, _row_spec, _w_spec],
    out_specs=_row_spec, out_shape=_out_shape)

_k2 = pl.pallas_call(
    _k2_body, grid=(N // BN,),
    in_specs=[_cnt_spec, _z_spec, _row_spec, _b_spec, _w_spec],
    out_specs=_row_spec, out_shape=_out_shape)

_k3 = pl.pallas_call(
    _k3_body, grid=(N // BN,),
    in_specs=[_cnt_spec, _z_spec, _row_spec, _b_spec],
    out_specs=_row_spec, out_shape=_out_shape)


def kernel(x, edge_index, W1, b1, W2, b2):
    ones16 = jnp.ones((K, 16), jnp.float32)
    z16 = jnp.zeros((RPT, 16), jnp.float32)
    zrows = jnp.zeros((RPT, D), jnp.float32)

    cnt = _cnt_kernel(edge_index, ones16, z16)
    y1 = _k1(cnt, x, W1)
    z1 = _edge_kernel(y1, edge_index, zrows)
    y2 = _k2(cnt, z1, y1, b1.reshape(1, D), W2)
    z2 = _edge_kernel(y2, edge_index, zrows)
    return _k3(cnt, z2, y2, b2.reshape(1, D))
